# baseline (speedup 1.0000x reference)
import functools

import jax
import jax.numpy as jnp
from jax import lax
from jax.experimental import pallas as pl
from jax.experimental.pallas import tpu as pltpu
from jax.experimental.pallas import tpu_sc as plsc

_VOCAB = 1000000
_DIM = 64
_B = 16384
_L = 16
_PROW = 2 * _DIM

_info = plsc.get_sparse_core_info()
_NC = _info.num_cores
_NS = _info.num_subcores
_NW = _NC * _NS
_BPW = _B // _NW
_CH = 128
_RCH = 128
_NRC = _BPW // _RCH

_mesh = plsc.VectorSubcoreMesh(core_axis_name="c", subcore_axis_name="s")

_GATHER_DN = lax.GatherDimensionNumbers(
    offset_dims=(), collapsed_slice_dims=(0,), start_index_map=(0,))


def _shuffle(x, perm):
    return lax.gather(x, perm[:, None], _GATHER_DN, slice_sizes=(1,),
                      mode=lax.GatherScatterMode.PROMISE_IN_BOUNDS)


@functools.partial(
    pl.kernel,
    mesh=_mesh,
    out_type=jax.ShapeDtypeStruct((_B,), jnp.float32),
    scratch_types=[
        pltpu.VMEM((_BPW,), jnp.int32),
        pltpu.VMEM((_BPW,), jnp.int32),
        pltpu.VMEM((_BPW,), jnp.int32),
        pltpu.VMEM((_BPW,), jnp.int32),
        pltpu.VMEM((2, _RCH, _PROW), jnp.float32),
        pltpu.VMEM((2, _RCH, _PROW), jnp.float32),
        pltpu.VMEM((_BPW,), jnp.float32),
        pltpu.SemaphoreType.DMA,
        pltpu.SemaphoreType.DMA,
    ],
)
def _sim_kernel(cidx_hbm, tidx_hbm, pairs_hbm, sim_hbm,
                cidx_v, tidx_v, crow_v, trow_v, cbuf_v, tbuf_v, sim_v,
                sem0, sem1):
    wid = lax.axis_index("s") * _NC + lax.axis_index("c")
    base = wid * _BPW

    pltpu.sync_copy(cidx_hbm.at[pl.ds(base, _BPW)], cidx_v)
    pltpu.sync_copy(tidx_hbm.at[pl.ds(base, _BPW)], tidx_v)

    def shift_body(g, carry):
        crow_v[pl.ds(g * _L, _L)] = lax.shift_right_logical(
            cidx_v[pl.ds(g * _L, _L)], 1)
        trow_v[pl.ds(g * _L, _L)] = lax.shift_right_logical(
            tidx_v[pl.ds(g * _L, _L)], 1)
        return carry
    lax.fori_loop(0, _BPW // _L, shift_body, 0)

    sems = (sem0, sem1)

    def fire(rc, slot):
        cps = []
        for k in range(_RCH // _CH):
            off = rc * _RCH + k * _CH
            cps.append(pltpu.async_copy(
                pairs_hbm.at[crow_v.at[pl.ds(off, _CH)]],
                cbuf_v.at[slot, pl.ds(k * _CH, _CH), :], sems[slot]))
            cps.append(pltpu.async_copy(
                pairs_hbm.at[trow_v.at[pl.ds(off, _CH)]],
                tbuf_v.at[slot, pl.ds(k * _CH, _CH), :], sems[slot]))
        return cps

    iota = jnp.arange(_L, dtype=jnp.int32)
    perms = [iota ^ h for h in (8, 4, 2, 1)]

    def compute_chunk(rc, slot):
        def group_body(g, carry):
            sim_g = jnp.zeros((_L,), jnp.float32)
            coff = rc * _RCH + g * _L
            civ = cidx_v[pl.ds(coff, _L)]
            tiv = tidx_v[pl.ds(coff, _L)]
            for j in range(_L):
                r = g * _L + j
                pc = (civ[j] & 1) * _DIM
                pt = (tiv[j] & 1) * _DIM
                p = (cbuf_v[slot, r, pl.ds(pc, _L)] *
                     tbuf_v[slot, r, pl.ds(pt, _L)])
                for k in range(1, _DIM // _L):
                    p = p + (cbuf_v[slot, r, pl.ds(pc + k * _L, _L)] *
                             tbuf_v[slot, r, pl.ds(pt + k * _L, _L)])
                for perm in perms:
                    p = p + _shuffle(p, perm)
                sim_g = jnp.where(iota == j, p, sim_g)
            sim_v[pl.ds(coff, _L)] = sim_g
            return carry
        lax.fori_loop(0, _RCH // _L, group_body, 0)

    pending = fire(0, 0)
    for rc in range(_NRC):
        nxt = []
        if rc + 1 < _NRC:
            nxt = fire(rc + 1, (rc + 1) % 2)
        for cp in pending:
            cp.wait()
        compute_chunk(rc, rc % 2)
        pending = nxt

    pltpu.sync_copy(sim_v, sim_hbm.at[pl.ds(base, _BPW)])


def _loss_body(sim_ref, label_ref, out_ref):
    s = sim_ref[...]
    lbl = label_ref[...]
    term = jnp.maximum(s, 0.0) - s * lbl + jnp.log1p(jnp.exp(-jnp.abs(s)))
    out_ref[0, 0] = jnp.sum(term)


def kernel(center_idx, target_idx, label, emb_weight, out_emb_weight):
    del out_emb_weight
    pairs = emb_weight.reshape(_VOCAB // 2, _PROW)
    sim = _sim_kernel(center_idx, target_idx, pairs)
    loss = pl.pallas_call(
        _loss_body,
        out_shape=jax.ShapeDtypeStruct((1, 1), jnp.float32),
        out_specs=pl.BlockSpec(memory_space=pltpu.SMEM),
    )(sim.reshape(128, 128), label.reshape(128, 128))
    return loss[0, 0]
